# baseline (speedup 1.0000x reference)
import functools

import jax
import jax.numpy as jnp
from jax import lax
from jax.experimental import pallas as pl
from jax.experimental.pallas import tpu as pltpu
from jax.experimental.pallas import tpu_sc as plsc

N = 20000
NF = 40000
NP = 20480
NFP = 40960
BQ = 1024
BDB = 2048
BIG = 1e30


def _nn_body(q_ref, db_ref, omin_ref, oidx_ref):
    q = q_ref[...]

    def step(c, carry):
        cmin, cidx = carry
        m = db_ref[:, pl.ds(c * BDB, BDB)]
        d = lax.dot_general(q, m, (((1,), (0,)), ((), ())),
                            preferred_element_type=jnp.float32
                            ).astype(jnp.bfloat16)
        for j in range(BDB // 128):
            dj = d[:, j * 128:(j + 1) * 128]
            slotv = jnp.full((BQ, 128), (c * (BDB // 128) + j),
                             jnp.bfloat16)
            upd = dj < cmin
            cmin = jnp.where(upd, dj, cmin)
            cidx = jnp.where(upd, slotv, cidx)
        return cmin, cidx

    init = (jnp.full((BQ, 128), jnp.inf, jnp.bfloat16),
            jnp.zeros((BQ, 128), jnp.bfloat16))
    cmin, cidx = lax.fori_loop(0, NP // BDB, step, init)
    cminf = cmin.astype(jnp.float32)
    rmin = jnp.min(cminf, axis=1, keepdims=True)
    lane = lax.broadcasted_iota(jnp.int32, (BQ, 128), 1)
    col = cidx.astype(jnp.float32).astype(jnp.int32) * 128 + lane
    ridx = jnp.min(jnp.where(cminf == rmin, col, jnp.int32(2**30)),
                   axis=1, keepdims=True)
    omin_ref[...] = rmin
    oidx_ref[...] = ridx


def _nn_call(q_bf, db_bf):
    return pl.pallas_call(
        _nn_body,
        grid=(NP // BQ,),
        in_specs=[
            pl.BlockSpec((BQ, 8), lambda i: (i, 0)),
            pl.BlockSpec((8, NP), lambda i: (0, 0)),
        ],
        out_specs=[
            pl.BlockSpec((BQ, 1), lambda i: (i, 0)),
            pl.BlockSpec((BQ, 1), lambda i: (i, 0)),
        ],
        out_shape=[
            jax.ShapeDtypeStruct((NP, 1), jnp.float32),
            jax.ShapeDtypeStruct((NP, 1), jnp.int32),
        ],
        compiler_params=pltpu.CompilerParams(
            dimension_semantics=("parallel",)),
    )(q_bf, db_bf)


_NC = 2
_NS = 16
_NW = _NC * _NS
_CH = 128


def _face_gather_body(sv_hbm, f0_hbm, f1_hbm, f2_hbm,
                      ov0_hbm, ov1_hbm, ov2_hbm, idx_v, rows_v, sem):
    wid = lax.axis_index("s") * _NC + lax.axis_index("c")
    nper = NFP // _NW
    base = wid * nper
    for idx_hbm, out_hbm in ((f0_hbm, ov0_hbm), (f1_hbm, ov1_hbm),
                             (f2_hbm, ov2_hbm)):
        pltpu.sync_copy(idx_hbm.at[pl.ds(base, nper)], idx_v.at[pl.ds(0, nper)])
        for j in range(nper // _CH):
            pltpu.async_copy(
                sv_hbm.at[idx_v.at[pl.ds(j * _CH, _CH)]],
                rows_v,
                sem,
            ).wait()
            pltpu.sync_copy(rows_v, out_hbm.at[pl.ds(base + j * _CH, _CH)])


@functools.lru_cache(maxsize=1)
def _face_gather_fn():
    return pl.kernel(
        _face_gather_body,
        mesh=plsc.VectorSubcoreMesh(core_axis_name="c", subcore_axis_name="s",
                                    num_cores=_NC, num_subcores=_NS),
        out_type=[
            jax.ShapeDtypeStruct((NFP, 4), jnp.float32),
            jax.ShapeDtypeStruct((NFP, 4), jnp.float32),
            jax.ShapeDtypeStruct((NFP, 4), jnp.float32),
        ],
        scratch_types=[
            pltpu.VMEM((NFP // _NW,), jnp.int32),
            pltpu.VMEM((_CH, 4), jnp.float32),
            pltpu.SemaphoreType.DMA,
        ],
        compiler_params=pltpu.CompilerParams(use_tc_tiling_on_sc=False),
    )


def _norm_gather_body(tab_hbm, idx_hbm, out_hbm, idx_v, rows_v, sem):
    wid = lax.axis_index("s") * _NC + lax.axis_index("c")
    nper = NP // _NW
    base = wid * nper
    pltpu.sync_copy(idx_hbm.at[pl.ds(base, nper)], idx_v.at[pl.ds(0, nper)])
    for j in range(nper // _CH):
        pltpu.async_copy(
            tab_hbm.at[idx_v.at[pl.ds(j * _CH, _CH)]],
            rows_v,
            sem,
        ).wait()
        pltpu.sync_copy(rows_v, out_hbm.at[pl.ds(base + j * _CH, _CH)])


@functools.lru_cache(maxsize=1)
def _norm_gather_fn():
    return pl.kernel(
        _norm_gather_body,
        mesh=plsc.VectorSubcoreMesh(core_axis_name="c", subcore_axis_name="s",
                                    num_cores=_NC, num_subcores=_NS),
        out_type=jax.ShapeDtypeStruct((NP, 4), jnp.float32),
        scratch_types=[
            pltpu.VMEM((NP // _NW,), jnp.int32),
            pltpu.VMEM((_CH, 4), jnp.float32),
            pltpu.SemaphoreType.DMA,
        ],
        compiler_params=pltpu.CompilerParams(use_tc_tiling_on_sc=False),
    )


def _fin_body(dxy_ref, dyx_ref, sn_ref, tn_ref, gxy_ref, gyx_ref,
              v0_ref, v1_ref, v2_ref, out_ref):
    col_n = lax.broadcasted_iota(jnp.int32, (1, NP), 1)
    mask_n = (col_n < N).astype(jnp.float32)
    inv_n = jnp.float32(1.0 / N)

    cham_dist = (jnp.sum(dxy_ref[...] * mask_n)
                 + jnp.sum(dyx_ref[...] * mask_n)) * inv_n

    def cos_dist(a, b):
        num = jnp.sum(a * b, axis=0, keepdims=True)
        den = (jnp.sqrt(jnp.sum(a * a, axis=0, keepdims=True))
               * jnp.sqrt(jnp.sum(b * b, axis=0, keepdims=True)) + 1e-8)
        return jnp.sum((1.0 - jnp.abs(num / den)) * mask_n) * inv_n

    cham_norm = (cos_dist(sn_ref[...][:3], gxy_ref[...][:3])
                 + cos_dist(tn_ref[...][:3], gyx_ref[...][:3]))
    loss_chamfer = cham_dist * 0.8 + cham_norm * 0.2

    col_f = lax.broadcasted_iota(jnp.int32, (1, NFP), 1)
    mask_f = (col_f < NF).astype(jnp.float32)
    inv_f = jnp.float32(1.0 / NF)
    v0 = v0_ref[...][:3]
    v1 = v1_ref[...][:3]
    v2 = v2_ref[...][:3]

    def elen(a, b):
        d = a - b
        return jnp.sqrt(jnp.sum(d * d, axis=0, keepdims=True))

    e0 = elen(v0, v1)
    e1 = elen(v1, v2)
    e2 = elen(v2, v0)
    fmean = lambda x: jnp.sum(x * mask_f) * inv_f
    avg_len = fmean(e0 + e1 + e2) / 3.0
    l1 = lambda a, b: fmean(jnp.abs(a - b))
    loss_face = (l1(e0, e1) + l1(e1, e2) + l1(e2, e0)
                 + l1(e0, avg_len) + l1(e1, avg_len) + l1(e2, avg_len))
    loss_edge = (fmean((e0 - avg_len) ** 2) + fmean((e1 - avg_len) ** 2)
                 + fmean((e2 - avg_len) ** 2)) / 3.0

    total = loss_chamfer * 0.8 + loss_edge * 0.2 + loss_face * 0.1
    out_ref[...] = jnp.reshape(total, (1, 1))


def _fin_call(dxy, dyx, sn4, tn4, gxy, gyx, v0, v1, v2):
    return pl.pallas_call(
        _fin_body,
        out_shape=jax.ShapeDtypeStruct((1, 1), jnp.float32),
    )(dxy, dyx, sn4, tn4, gxy, gyx, v0, v1, v2)


def _hilo(x):
    hi = x.astype(jnp.bfloat16).astype(jnp.float32)
    return hi, x - hi


def _prep_queries(v):
    sq = jnp.sum(v * v, axis=1, keepdims=True)
    hi, lo = _hilo(sq)
    a = jnp.concatenate([v, jnp.ones((N, 2), jnp.float32), hi, lo], axis=1)
    return jnp.pad(a, ((0, NP - N), (0, 1))).astype(jnp.bfloat16)


def _prep_db(v):
    sq = jnp.pad(jnp.sum(v * v, axis=1), (0, NP - N), constant_values=BIG)
    hi, lo = _hilo(sq)
    ones = jnp.ones((2, NP), jnp.float32)
    a = jnp.concatenate([jnp.pad(-2.0 * v.T, ((0, 0), (0, NP - N))),
                         hi[None, :], lo[None, :], ones], axis=0)
    return jnp.pad(a, ((0, 1), (0, 0))).astype(jnp.bfloat16)


def _pad4(t):
    return jnp.pad(t, ((0, 0), (0, 1)))


def kernel(src_verts, src_normals, trg_verts, trg_normals, faces):
    sq_bf = _prep_queries(src_verts)
    tq_bf = _prep_queries(trg_verts)
    sdb_bf = _prep_db(src_verts)
    tdb_bf = _prep_db(trg_verts)

    fpad = jnp.pad(faces, ((0, NFP - NF), (0, 0)))
    gv0, gv1, gv2 = _face_gather_fn()(
        _pad4(src_verts), fpad[:, 0], fpad[:, 1], fpad[:, 2])

    dxy, ixy = _nn_call(sq_bf, tdb_bf)
    gxy = _norm_gather_fn()(_pad4(trg_normals), ixy.reshape(NP))
    dyx, iyx = _nn_call(tq_bf, sdb_bf)
    gyx = _norm_gather_fn()(_pad4(src_normals), iyx.reshape(NP))

    sn_t = jnp.pad(src_normals.T, ((0, 0), (0, NP - N)))
    tn_t = jnp.pad(trg_normals.T, ((0, 0), (0, NP - N)))
    out = _fin_call(dxy.T, dyx.T, sn_t, tn_t, gxy.T, gyx.T,
                    gv0.T, gv1.T, gv2.T)
    return out.reshape(())


# speedup vs baseline: 1.0467x; 1.0467x over previous
import functools

import jax
import jax.numpy as jnp
from jax import lax
from jax.experimental import pallas as pl
from jax.experimental.pallas import tpu as pltpu
from jax.experimental.pallas import tpu_sc as plsc

N = 20000
NF = 40000
NP = 20480
NFP = 40960
BQ = 1024
BDB = 4096
BIG = 1e30


def _nn_body(q_ref, db_ref, omin_ref, oidx_ref):
    q = q_ref[...]

    def step(c, carry):
        cmin, cidx = carry
        m = db_ref[:, pl.ds(c * BDB, BDB)]
        d = lax.dot_general(q, m, (((1,), (0,)), ((), ())),
                            preferred_element_type=jnp.float32
                            ).astype(jnp.bfloat16)
        for j in range(BDB // 128):
            dj = d[:, j * 128:(j + 1) * 128]
            slotv = jnp.full((BQ, 128), (c * (BDB // 128) + j),
                             jnp.bfloat16)
            upd = dj < cmin
            cmin = jnp.where(upd, dj, cmin)
            cidx = jnp.where(upd, slotv, cidx)
        return cmin, cidx

    init = (jnp.full((BQ, 128), jnp.inf, jnp.bfloat16),
            jnp.zeros((BQ, 128), jnp.bfloat16))
    cmin, cidx = lax.fori_loop(0, NP // BDB, step, init)
    cminf = cmin.astype(jnp.float32)
    rmin = jnp.min(cminf, axis=1, keepdims=True)
    lane = lax.broadcasted_iota(jnp.int32, (BQ, 128), 1)
    col = cidx.astype(jnp.float32).astype(jnp.int32) * 128 + lane
    ridx = jnp.min(jnp.where(cminf == rmin, col, jnp.int32(2**30)),
                   axis=1, keepdims=True)
    omin_ref[...] = rmin
    oidx_ref[...] = ridx


def _nn_call(q_bf, db_bf):
    return pl.pallas_call(
        _nn_body,
        grid=(NP // BQ,),
        in_specs=[
            pl.BlockSpec((BQ, 8), lambda i: (i, 0)),
            pl.BlockSpec((8, NP), lambda i: (0, 0)),
        ],
        out_specs=[
            pl.BlockSpec((BQ, 1), lambda i: (i, 0)),
            pl.BlockSpec((BQ, 1), lambda i: (i, 0)),
        ],
        out_shape=[
            jax.ShapeDtypeStruct((NP, 1), jnp.float32),
            jax.ShapeDtypeStruct((NP, 1), jnp.int32),
        ],
        compiler_params=pltpu.CompilerParams(
            dimension_semantics=("parallel",)),
    )(q_bf, db_bf)


_NC = 2
_NS = 16
_NW = _NC * _NS
_CH = 128


def _face_gather_body(sv_hbm, f0_hbm, f1_hbm, f2_hbm,
                      ov0_hbm, ov1_hbm, ov2_hbm, idx_v, rows_v, sem):
    wid = lax.axis_index("s") * _NC + lax.axis_index("c")
    nper = NFP // _NW
    base = wid * nper
    for idx_hbm, out_hbm in ((f0_hbm, ov0_hbm), (f1_hbm, ov1_hbm),
                             (f2_hbm, ov2_hbm)):
        pltpu.sync_copy(idx_hbm.at[pl.ds(base, nper)], idx_v.at[pl.ds(0, nper)])
        for j in range(nper // _CH):
            pltpu.async_copy(
                sv_hbm.at[idx_v.at[pl.ds(j * _CH, _CH)]],
                rows_v,
                sem,
            ).wait()
            pltpu.sync_copy(rows_v, out_hbm.at[pl.ds(base + j * _CH, _CH)])


@functools.lru_cache(maxsize=1)
def _face_gather_fn():
    return pl.kernel(
        _face_gather_body,
        mesh=plsc.VectorSubcoreMesh(core_axis_name="c", subcore_axis_name="s",
                                    num_cores=_NC, num_subcores=_NS),
        out_type=[
            jax.ShapeDtypeStruct((NFP, 4), jnp.float32),
            jax.ShapeDtypeStruct((NFP, 4), jnp.float32),
            jax.ShapeDtypeStruct((NFP, 4), jnp.float32),
        ],
        scratch_types=[
            pltpu.VMEM((NFP // _NW,), jnp.int32),
            pltpu.VMEM((_CH, 4), jnp.float32),
            pltpu.SemaphoreType.DMA,
        ],
        compiler_params=pltpu.CompilerParams(use_tc_tiling_on_sc=False),
    )


def _norm_gather_body(tab_hbm, idx_hbm, out_hbm, idx_v, rows_v, sem):
    wid = lax.axis_index("s") * _NC + lax.axis_index("c")
    nper = NP // _NW
    base = wid * nper
    pltpu.sync_copy(idx_hbm.at[pl.ds(base, nper)], idx_v.at[pl.ds(0, nper)])
    for j in range(nper // _CH):
        pltpu.async_copy(
            tab_hbm.at[idx_v.at[pl.ds(j * _CH, _CH)]],
            rows_v,
            sem,
        ).wait()
        pltpu.sync_copy(rows_v, out_hbm.at[pl.ds(base + j * _CH, _CH)])


@functools.lru_cache(maxsize=1)
def _norm_gather_fn():
    return pl.kernel(
        _norm_gather_body,
        mesh=plsc.VectorSubcoreMesh(core_axis_name="c", subcore_axis_name="s",
                                    num_cores=_NC, num_subcores=_NS),
        out_type=jax.ShapeDtypeStruct((NP, 4), jnp.float32),
        scratch_types=[
            pltpu.VMEM((NP // _NW,), jnp.int32),
            pltpu.VMEM((_CH, 4), jnp.float32),
            pltpu.SemaphoreType.DMA,
        ],
        compiler_params=pltpu.CompilerParams(use_tc_tiling_on_sc=False),
    )


def _fin_body(dxy_ref, dyx_ref, sn_ref, tn_ref, gxy_ref, gyx_ref,
              v0_ref, v1_ref, v2_ref, out_ref):
    col_n = lax.broadcasted_iota(jnp.int32, (1, NP), 1)
    mask_n = (col_n < N).astype(jnp.float32)
    inv_n = jnp.float32(1.0 / N)

    cham_dist = (jnp.sum(dxy_ref[...] * mask_n)
                 + jnp.sum(dyx_ref[...] * mask_n)) * inv_n

    def cos_dist(a, b):
        num = jnp.sum(a * b, axis=0, keepdims=True)
        den = (jnp.sqrt(jnp.sum(a * a, axis=0, keepdims=True))
               * jnp.sqrt(jnp.sum(b * b, axis=0, keepdims=True)) + 1e-8)
        return jnp.sum((1.0 - jnp.abs(num / den)) * mask_n) * inv_n

    cham_norm = (cos_dist(sn_ref[...][:3], gxy_ref[...][:3])
                 + cos_dist(tn_ref[...][:3], gyx_ref[...][:3]))
    loss_chamfer = cham_dist * 0.8 + cham_norm * 0.2

    col_f = lax.broadcasted_iota(jnp.int32, (1, NFP), 1)
    mask_f = (col_f < NF).astype(jnp.float32)
    inv_f = jnp.float32(1.0 / NF)
    v0 = v0_ref[...][:3]
    v1 = v1_ref[...][:3]
    v2 = v2_ref[...][:3]

    def elen(a, b):
        d = a - b
        return jnp.sqrt(jnp.sum(d * d, axis=0, keepdims=True))

    e0 = elen(v0, v1)
    e1 = elen(v1, v2)
    e2 = elen(v2, v0)
    fmean = lambda x: jnp.sum(x * mask_f) * inv_f
    avg_len = fmean(e0 + e1 + e2) / 3.0
    l1 = lambda a, b: fmean(jnp.abs(a - b))
    loss_face = (l1(e0, e1) + l1(e1, e2) + l1(e2, e0)
                 + l1(e0, avg_len) + l1(e1, avg_len) + l1(e2, avg_len))
    loss_edge = (fmean((e0 - avg_len) ** 2) + fmean((e1 - avg_len) ** 2)
                 + fmean((e2 - avg_len) ** 2)) / 3.0

    total = loss_chamfer * 0.8 + loss_edge * 0.2 + loss_face * 0.1
    out_ref[...] = jnp.reshape(total, (1, 1))


def _fin_call(dxy, dyx, sn4, tn4, gxy, gyx, v0, v1, v2):
    return pl.pallas_call(
        _fin_body,
        out_shape=jax.ShapeDtypeStruct((1, 1), jnp.float32),
    )(dxy, dyx, sn4, tn4, gxy, gyx, v0, v1, v2)


def _hilo(x):
    hi = x.astype(jnp.bfloat16).astype(jnp.float32)
    return hi, x - hi


def _prep_queries(v):
    sq = jnp.sum(v * v, axis=1, keepdims=True)
    hi, lo = _hilo(sq)
    a = jnp.concatenate([v, jnp.ones((N, 2), jnp.float32), hi, lo], axis=1)
    return jnp.pad(a, ((0, NP - N), (0, 1))).astype(jnp.bfloat16)


def _prep_db(v):
    sq = jnp.pad(jnp.sum(v * v, axis=1), (0, NP - N), constant_values=BIG)
    hi, lo = _hilo(sq)
    ones = jnp.ones((2, NP), jnp.float32)
    a = jnp.concatenate([jnp.pad(-2.0 * v.T, ((0, 0), (0, NP - N))),
                         hi[None, :], lo[None, :], ones], axis=0)
    return jnp.pad(a, ((0, 1), (0, 0))).astype(jnp.bfloat16)


def _pad4(t):
    return jnp.pad(t, ((0, 0), (0, 1)))


def kernel(src_verts, src_normals, trg_verts, trg_normals, faces):
    sq_bf = _prep_queries(src_verts)
    tq_bf = _prep_queries(trg_verts)
    sdb_bf = _prep_db(src_verts)
    tdb_bf = _prep_db(trg_verts)

    fpad = jnp.pad(faces, ((0, NFP - NF), (0, 0)))
    gv0, gv1, gv2 = _face_gather_fn()(
        _pad4(src_verts), fpad[:, 0], fpad[:, 1], fpad[:, 2])

    dxy, ixy = _nn_call(sq_bf, tdb_bf)
    gxy = _norm_gather_fn()(_pad4(trg_normals), ixy.reshape(NP))
    dyx, iyx = _nn_call(tq_bf, sdb_bf)
    gyx = _norm_gather_fn()(_pad4(src_normals), iyx.reshape(NP))

    sn_t = jnp.pad(src_normals.T, ((0, 0), (0, NP - N)))
    tn_t = jnp.pad(trg_normals.T, ((0, 0), (0, NP - N)))
    out = _fin_call(dxy.T, dyx.T, sn_t, tn_t, gxy.T, gyx.T,
                    gv0.T, gv1.T, gv2.T)
    return out.reshape(())


# speedup vs baseline: 1.0567x; 1.0095x over previous
import functools

import jax
import jax.numpy as jnp
from jax import lax
from jax.experimental import pallas as pl
from jax.experimental.pallas import tpu as pltpu
from jax.experimental.pallas import tpu_sc as plsc

N = 20000
NF = 40000
NP = 20480
NFP = 40960
BQ = 1024
BDB = 5120
BIG = 1e30


def _nn_body(q_ref, db_ref, omin_ref, oidx_ref):
    q = q_ref[...]

    def step(c, carry):
        cmin, cidx = carry
        m = db_ref[:, pl.ds(c * BDB, BDB)]
        d = lax.dot_general(q, m, (((1,), (0,)), ((), ())),
                            preferred_element_type=jnp.float32
                            ).astype(jnp.bfloat16)
        for j in range(BDB // 128):
            dj = d[:, j * 128:(j + 1) * 128]
            slotv = jnp.full((BQ, 128), (c * (BDB // 128) + j),
                             jnp.bfloat16)
            upd = dj < cmin
            cmin = jnp.where(upd, dj, cmin)
            cidx = jnp.where(upd, slotv, cidx)
        return cmin, cidx

    init = (jnp.full((BQ, 128), jnp.inf, jnp.bfloat16),
            jnp.zeros((BQ, 128), jnp.bfloat16))
    cmin, cidx = lax.fori_loop(0, NP // BDB, step, init)
    cminf = cmin.astype(jnp.float32)
    rmin = jnp.min(cminf, axis=1, keepdims=True)
    lane = lax.broadcasted_iota(jnp.int32, (BQ, 128), 1)
    col = cidx.astype(jnp.float32).astype(jnp.int32) * 128 + lane
    ridx = jnp.min(jnp.where(cminf == rmin, col, jnp.int32(2**30)),
                   axis=1, keepdims=True)
    omin_ref[...] = rmin
    oidx_ref[...] = ridx


def _nn_call(q_bf, db_bf):
    return pl.pallas_call(
        _nn_body,
        grid=(NP // BQ,),
        in_specs=[
            pl.BlockSpec((BQ, 8), lambda i: (i, 0)),
            pl.BlockSpec((8, NP), lambda i: (0, 0)),
        ],
        out_specs=[
            pl.BlockSpec((BQ, 1), lambda i: (i, 0)),
            pl.BlockSpec((BQ, 1), lambda i: (i, 0)),
        ],
        out_shape=[
            jax.ShapeDtypeStruct((NP, 1), jnp.float32),
            jax.ShapeDtypeStruct((NP, 1), jnp.int32),
        ],
        compiler_params=pltpu.CompilerParams(
            dimension_semantics=("parallel",)),
    )(q_bf, db_bf)


_NC = 2
_NS = 16
_NW = _NC * _NS
_CH = 128


def _face_gather_body(sv_hbm, f0_hbm, f1_hbm, f2_hbm,
                      ov0_hbm, ov1_hbm, ov2_hbm, idx_v, rows_v, sem):
    wid = lax.axis_index("s") * _NC + lax.axis_index("c")
    nper = NFP // _NW
    base = wid * nper
    for idx_hbm, out_hbm in ((f0_hbm, ov0_hbm), (f1_hbm, ov1_hbm),
                             (f2_hbm, ov2_hbm)):
        pltpu.sync_copy(idx_hbm.at[pl.ds(base, nper)], idx_v.at[pl.ds(0, nper)])
        for j in range(nper // _CH):
            pltpu.async_copy(
                sv_hbm.at[idx_v.at[pl.ds(j * _CH, _CH)]],
                rows_v,
                sem,
            ).wait()
            pltpu.sync_copy(rows_v, out_hbm.at[pl.ds(base + j * _CH, _CH)])


@functools.lru_cache(maxsize=1)
def _face_gather_fn():
    return pl.kernel(
        _face_gather_body,
        mesh=plsc.VectorSubcoreMesh(core_axis_name="c", subcore_axis_name="s",
                                    num_cores=_NC, num_subcores=_NS),
        out_type=[
            jax.ShapeDtypeStruct((NFP, 4), jnp.float32),
            jax.ShapeDtypeStruct((NFP, 4), jnp.float32),
            jax.ShapeDtypeStruct((NFP, 4), jnp.float32),
        ],
        scratch_types=[
            pltpu.VMEM((NFP // _NW,), jnp.int32),
            pltpu.VMEM((_CH, 4), jnp.float32),
            pltpu.SemaphoreType.DMA,
        ],
        compiler_params=pltpu.CompilerParams(use_tc_tiling_on_sc=False),
    )


def _norm_gather_body(tab_hbm, idx_hbm, out_hbm, idx_v, rows_v, sem):
    wid = lax.axis_index("s") * _NC + lax.axis_index("c")
    nper = NP // _NW
    base = wid * nper
    pltpu.sync_copy(idx_hbm.at[pl.ds(base, nper)], idx_v.at[pl.ds(0, nper)])
    for j in range(nper // _CH):
        pltpu.async_copy(
            tab_hbm.at[idx_v.at[pl.ds(j * _CH, _CH)]],
            rows_v,
            sem,
        ).wait()
        pltpu.sync_copy(rows_v, out_hbm.at[pl.ds(base + j * _CH, _CH)])


@functools.lru_cache(maxsize=1)
def _norm_gather_fn():
    return pl.kernel(
        _norm_gather_body,
        mesh=plsc.VectorSubcoreMesh(core_axis_name="c", subcore_axis_name="s",
                                    num_cores=_NC, num_subcores=_NS),
        out_type=jax.ShapeDtypeStruct((NP, 4), jnp.float32),
        scratch_types=[
            pltpu.VMEM((NP // _NW,), jnp.int32),
            pltpu.VMEM((_CH, 4), jnp.float32),
            pltpu.SemaphoreType.DMA,
        ],
        compiler_params=pltpu.CompilerParams(use_tc_tiling_on_sc=False),
    )


def _fin_body(dxy_ref, dyx_ref, sn_ref, tn_ref, gxy_ref, gyx_ref,
              v0_ref, v1_ref, v2_ref, out_ref):
    col_n = lax.broadcasted_iota(jnp.int32, (1, NP), 1)
    mask_n = (col_n < N).astype(jnp.float32)
    inv_n = jnp.float32(1.0 / N)

    cham_dist = (jnp.sum(dxy_ref[...] * mask_n)
                 + jnp.sum(dyx_ref[...] * mask_n)) * inv_n

    def cos_dist(a, b):
        num = jnp.sum(a * b, axis=0, keepdims=True)
        den = (jnp.sqrt(jnp.sum(a * a, axis=0, keepdims=True))
               * jnp.sqrt(jnp.sum(b * b, axis=0, keepdims=True)) + 1e-8)
        return jnp.sum((1.0 - jnp.abs(num / den)) * mask_n) * inv_n

    cham_norm = (cos_dist(sn_ref[...][:3], gxy_ref[...][:3])
                 + cos_dist(tn_ref[...][:3], gyx_ref[...][:3]))
    loss_chamfer = cham_dist * 0.8 + cham_norm * 0.2

    col_f = lax.broadcasted_iota(jnp.int32, (1, NFP), 1)
    mask_f = (col_f < NF).astype(jnp.float32)
    inv_f = jnp.float32(1.0 / NF)
    v0 = v0_ref[...][:3]
    v1 = v1_ref[...][:3]
    v2 = v2_ref[...][:3]

    def elen(a, b):
        d = a - b
        return jnp.sqrt(jnp.sum(d * d, axis=0, keepdims=True))

    e0 = elen(v0, v1)
    e1 = elen(v1, v2)
    e2 = elen(v2, v0)
    fmean = lambda x: jnp.sum(x * mask_f) * inv_f
    avg_len = fmean(e0 + e1 + e2) / 3.0
    l1 = lambda a, b: fmean(jnp.abs(a - b))
    loss_face = (l1(e0, e1) + l1(e1, e2) + l1(e2, e0)
                 + l1(e0, avg_len) + l1(e1, avg_len) + l1(e2, avg_len))
    loss_edge = (fmean((e0 - avg_len) ** 2) + fmean((e1 - avg_len) ** 2)
                 + fmean((e2 - avg_len) ** 2)) / 3.0

    total = loss_chamfer * 0.8 + loss_edge * 0.2 + loss_face * 0.1
    out_ref[...] = jnp.reshape(total, (1, 1))


def _fin_call(dxy, dyx, sn4, tn4, gxy, gyx, v0, v1, v2):
    return pl.pallas_call(
        _fin_body,
        out_shape=jax.ShapeDtypeStruct((1, 1), jnp.float32),
    )(dxy, dyx, sn4, tn4, gxy, gyx, v0, v1, v2)


def _hilo(x):
    hi = x.astype(jnp.bfloat16).astype(jnp.float32)
    return hi, x - hi


def _prep_queries(v):
    sq = jnp.sum(v * v, axis=1, keepdims=True)
    hi, lo = _hilo(sq)
    a = jnp.concatenate([v, jnp.ones((N, 2), jnp.float32), hi, lo], axis=1)
    return jnp.pad(a, ((0, NP - N), (0, 1))).astype(jnp.bfloat16)


def _prep_db(v):
    sq = jnp.pad(jnp.sum(v * v, axis=1), (0, NP - N), constant_values=BIG)
    hi, lo = _hilo(sq)
    ones = jnp.ones((2, NP), jnp.float32)
    a = jnp.concatenate([jnp.pad(-2.0 * v.T, ((0, 0), (0, NP - N))),
                         hi[None, :], lo[None, :], ones], axis=0)
    return jnp.pad(a, ((0, 1), (0, 0))).astype(jnp.bfloat16)


def _pad4(t):
    return jnp.pad(t, ((0, 0), (0, 1)))


def kernel(src_verts, src_normals, trg_verts, trg_normals, faces):
    sq_bf = _prep_queries(src_verts)
    tq_bf = _prep_queries(trg_verts)
    sdb_bf = _prep_db(src_verts)
    tdb_bf = _prep_db(trg_verts)

    fpad = jnp.pad(faces, ((0, NFP - NF), (0, 0)))
    gv0, gv1, gv2 = _face_gather_fn()(
        _pad4(src_verts), fpad[:, 0], fpad[:, 1], fpad[:, 2])

    dxy, ixy = _nn_call(sq_bf, tdb_bf)
    gxy = _norm_gather_fn()(_pad4(trg_normals), ixy.reshape(NP))
    dyx, iyx = _nn_call(tq_bf, sdb_bf)
    gyx = _norm_gather_fn()(_pad4(src_normals), iyx.reshape(NP))

    sn_t = jnp.pad(src_normals.T, ((0, 0), (0, NP - N)))
    tn_t = jnp.pad(trg_normals.T, ((0, 0), (0, NP - N)))
    out = _fin_call(dxy.T, dyx.T, sn_t, tn_t, gxy.T, gyx.T,
                    gv0.T, gv1.T, gv2.T)
    return out.reshape(())
